# baseline (speedup 1.0000x reference)
import functools

import jax
import jax.numpy as jnp
from jax import lax
from jax.experimental import pallas as pl
from jax.experimental.pallas import tpu as pltpu
from jax.experimental.pallas import tpu_sc as plsc

_NC = 2
_NS = 16
_NW = _NC * _NS
_L = 16

_CC = 1024
_U = 8


def _sc_scan_chunk(buf, acc, rows_idx):
    cc = buf.shape[1]

    def step(i, carry):
        acc, colv = carry
        for _ in range(_U):
            v = plsc.load_gather(buf, [rows_idx, colv])
            acc = acc + v
            plsc.store_scatter(buf, [rows_idx, colv], acc)
            colv = colv + 1
        return acc, colv

    colv0 = jnp.zeros((_L,), jnp.int32)
    acc, _ = lax.fori_loop(0, cc // _U, step, (acc, colv0))
    return acc


def _sc_body(x_hbm, o_hbm, buf):
    rows, cols = x_hbm.shape
    rows_per_w = rows // _NW
    wid = lax.axis_index("s") * _NC + lax.axis_index("c")
    row0 = wid * rows_per_w
    rows_idx = lax.iota(jnp.int32, _L)
    for g in range(rows_per_w // _L):
        r0 = row0 + g * _L
        acc = jnp.zeros((_L,), jnp.float32)
        for c in range(cols // _CC):
            src = x_hbm.at[pl.ds(r0, _L), pl.ds(c * _CC, _CC)]
            pltpu.sync_copy(src, buf)
            acc = _sc_scan_chunk(buf, acc, rows_idx)
            dst = o_hbm.at[pl.ds(r0, _L), pl.ds(c * _CC, _CC)]
            pltpu.sync_copy(buf, dst)


@jax.jit
def kernel(x):
    rows, cols = x.shape
    mesh = plsc.VectorSubcoreMesh(core_axis_name="c", subcore_axis_name="s")
    return pl.kernel(
        _sc_body,
        out_type=jax.ShapeDtypeStruct((rows, cols), x.dtype),
        mesh=mesh,
        scratch_types=[pltpu.VMEM((_L, _CC), jnp.float32)],
        compiler_params=pltpu.CompilerParams(use_tc_tiling_on_sc=False, needs_layout_passes=False),
    )(x)


def _tc_cumsum_body(x_ref, o_ref, *, cb: int):
    rb, cols = x_ref.shape
    nchunk = cols // cb
    row = lax.broadcasted_iota(jnp.int32, (cb, cb), 0)
    col = lax.broadcasted_iota(jnp.int32, (cb, cb), 1)
    tri = (row <= col).astype(jnp.float32)

    carry = jnp.zeros((rb, 1), jnp.float32)
    for c in range(nchunk):
        blk = x_ref[:, c * cb : (c + 1) * cb]
        cs = lax.dot(blk, tri, preferred_element_type=jnp.float32)
        o_ref[:, c * cb : (c + 1) * cb] = cs + carry
        carry = carry + cs[:, cb - 1 : cb]


@jax.jit
def kernel_tc(x):
    rows, cols = x.shape
    rb = 256
    cb = 256
    body = functools.partial(_tc_cumsum_body, cb=cb)
    return pl.pallas_call(
        body,
        grid=(rows // rb,),
        in_specs=[pl.BlockSpec((rb, cols), lambda i: (i, 0))],
        out_specs=pl.BlockSpec((rb, cols), lambda i: (i, 0)),
        out_shape=jax.ShapeDtypeStruct((rows, cols), x.dtype),
    )(x)


# speedup vs baseline: 1.1281x; 1.1281x over previous
import functools

import jax
import jax.numpy as jnp
from jax import lax
from jax.experimental import pallas as pl
from jax.experimental.pallas import tpu as pltpu
from jax.experimental.pallas import tpu_sc as plsc

_NC = 2
_NS = 16
_NW = _NC * _NS
_L = 16

_CC = 512
_U = 8


def _sc_scan_pair(in_a, in_b, out_a, out_b, acc_a, acc_b, rows_idx):
    cc = in_a.shape[1]

    def step(i, carry):
        acc_a, acc_b, colv = carry
        for _ in range(_U):
            va = plsc.load_gather(in_a, [rows_idx, colv])
            vb = plsc.load_gather(in_b, [rows_idx, colv])
            acc_a = acc_a + va
            acc_b = acc_b + vb
            plsc.store_scatter(out_a, [rows_idx, colv], acc_a)
            plsc.store_scatter(out_b, [rows_idx, colv], acc_b)
            colv = colv + 1
        return acc_a, acc_b, colv

    colv0 = jnp.zeros((_L,), jnp.int32)
    acc_a, acc_b, _ = lax.fori_loop(0, cc // _U, step, (acc_a, acc_b, colv0))
    return acc_a, acc_b


def _sc_body(x_hbm, o_hbm, in_bufs, out_bufs, in_sems, out_sems):
    rows, cols = x_hbm.shape
    rows_per_w = rows // _NW
    nch = cols // _CC
    wid = lax.axis_index("s") * _NC + lax.axis_index("c")
    row0 = wid * rows_per_w
    rows_idx = lax.iota(jnp.int32, _L)

    def start_in(c, p, ra, rb):
        pltpu.async_copy(
            x_hbm.at[pl.ds(ra, _L), pl.ds(c * _CC, _CC)], in_bufs[2 * p], in_sems[2 * p]
        )
        pltpu.async_copy(
            x_hbm.at[pl.ds(rb, _L), pl.ds(c * _CC, _CC)], in_bufs[2 * p + 1], in_sems[2 * p + 1]
        )

    def wait_in(c, p, ra, rb):
        pltpu.make_async_copy(
            x_hbm.at[pl.ds(ra, _L), pl.ds(c * _CC, _CC)], in_bufs[2 * p], in_sems[2 * p]
        ).wait()
        pltpu.make_async_copy(
            x_hbm.at[pl.ds(rb, _L), pl.ds(c * _CC, _CC)], in_bufs[2 * p + 1], in_sems[2 * p + 1]
        ).wait()

    def start_out(c, p, ra, rb):
        pltpu.async_copy(
            out_bufs[2 * p], o_hbm.at[pl.ds(ra, _L), pl.ds(c * _CC, _CC)], out_sems[2 * p]
        )
        pltpu.async_copy(
            out_bufs[2 * p + 1], o_hbm.at[pl.ds(rb, _L), pl.ds(c * _CC, _CC)], out_sems[2 * p + 1]
        )

    def wait_out(c, p, ra, rb):
        pltpu.make_async_copy(
            out_bufs[2 * p], o_hbm.at[pl.ds(ra, _L), pl.ds(c * _CC, _CC)], out_sems[2 * p]
        ).wait()
        pltpu.make_async_copy(
            out_bufs[2 * p + 1], o_hbm.at[pl.ds(rb, _L), pl.ds(c * _CC, _CC)], out_sems[2 * p + 1]
        ).wait()

    def do_pair(pair, _):
        ra = row0 + (2 * pair) * _L
        rb = ra + _L
        start_in(0, 0, ra, rb)
        acc_a = jnp.zeros((_L,), jnp.float32)
        acc_b = jnp.zeros((_L,), jnp.float32)
        for c in range(nch):
            p = c % 2
            if c + 1 < nch:
                start_in(c + 1, 1 - p, ra, rb)
            wait_in(c, p, ra, rb)
            if c >= 2:
                wait_out(c - 2, p, ra, rb)
            acc_a, acc_b = _sc_scan_pair(
                in_bufs[2 * p], in_bufs[2 * p + 1],
                out_bufs[2 * p], out_bufs[2 * p + 1],
                acc_a, acc_b, rows_idx,
            )
            start_out(c, p, ra, rb)
        wait_out(nch - 2, nch % 2, ra, rb)
        wait_out(nch - 1, (nch - 1) % 2, ra, rb)
        return _

    lax.fori_loop(0, rows_per_w // (2 * _L), do_pair, 0)


@jax.jit
def kernel(x):
    rows, cols = x.shape
    mesh = plsc.VectorSubcoreMesh(core_axis_name="c", subcore_axis_name="s")
    buf = pltpu.VMEM((_L, _CC), jnp.float32)
    return pl.kernel(
        _sc_body,
        out_type=jax.ShapeDtypeStruct((rows, cols), x.dtype),
        mesh=mesh,
        scratch_types=[
            [buf] * 4,
            [buf] * 4,
            [pltpu.SemaphoreType.DMA] * 4,
            [pltpu.SemaphoreType.DMA] * 4,
        ],
        compiler_params=pltpu.CompilerParams(
            use_tc_tiling_on_sc=False, needs_layout_passes=False
        ),
    )(x)


def _tc_cumsum_body(x_ref, o_ref, *, cb: int):
    rb, cols = x_ref.shape
    nchunk = cols // cb
    row = lax.broadcasted_iota(jnp.int32, (cb, cb), 0)
    col = lax.broadcasted_iota(jnp.int32, (cb, cb), 1)
    tri = (row <= col).astype(jnp.float32)

    carry = jnp.zeros((rb, 1), jnp.float32)
    for c in range(nchunk):
        blk = x_ref[:, c * cb : (c + 1) * cb]
        cs = lax.dot(blk, tri, preferred_element_type=jnp.float32)
        o_ref[:, c * cb : (c + 1) * cb] = cs + carry
        carry = carry + cs[:, cb - 1 : cb]


@jax.jit
def kernel_tc(x):
    rows, cols = x.shape
    rb = 256
    cb = 256
    body = functools.partial(_tc_cumsum_body, cb=cb)
    return pl.pallas_call(
        body,
        grid=(rows // rb,),
        in_specs=[pl.BlockSpec((rb, cols), lambda i: (i, 0))],
        out_specs=pl.BlockSpec((rb, cols), lambda i: (i, 0)),
        out_shape=jax.ShapeDtypeStruct((rows, cols), x.dtype),
    )(x)


# speedup vs baseline: 2.9918x; 2.6521x over previous
import functools

import jax
import jax.numpy as jnp
from jax import lax
from jax.experimental import pallas as pl
from jax.experimental.pallas import tpu as pltpu
from jax.experimental.pallas import tpu_sc as plsc

_NC = 2
_NS = 16
_NW = _NC * _NS
_L = 16

_CC = 512
_CCP = _CC + 1
_U = 8


def _sc_scan_pair(in_a, in_b, out_a, out_b, acc_a, acc_b, rows_idx):
    cc = _CC

    def step(i, carry):
        acc_a, acc_b, colv = carry
        for _ in range(_U):
            va = plsc.load_gather(in_a, [rows_idx, colv])
            vb = plsc.load_gather(in_b, [rows_idx, colv])
            acc_a = acc_a + va
            acc_b = acc_b + vb
            plsc.store_scatter(out_a, [rows_idx, colv], acc_a)
            plsc.store_scatter(out_b, [rows_idx, colv], acc_b)
            colv = colv + 1
        return acc_a, acc_b, colv

    colv0 = jnp.zeros((_L,), jnp.int32)
    acc_a, acc_b, _ = lax.fori_loop(0, cc // _U, step, (acc_a, acc_b, colv0))
    return acc_a, acc_b


def _sc_body(x_hbm, o_hbm, in_bufs, out_bufs, in_sems, out_sems):
    rows, cols = x_hbm.shape
    rows_per_w = rows // _NW
    nch = cols // _CC
    wid = lax.axis_index("s") * _NC + lax.axis_index("c")
    row0 = wid * rows_per_w
    rows_idx = lax.iota(jnp.int32, _L)

    def _in_view(i):
        return in_bufs[i].at[:, pl.ds(0, _CC)]

    def _out_view(i):
        return out_bufs[i].at[:, pl.ds(0, _CC)]

    def start_in(c, p, ra, rb):
        pltpu.async_copy(
            x_hbm.at[pl.ds(ra, _L), pl.ds(c * _CC, _CC)], _in_view(2 * p), in_sems[2 * p]
        )
        pltpu.async_copy(
            x_hbm.at[pl.ds(rb, _L), pl.ds(c * _CC, _CC)], _in_view(2 * p + 1), in_sems[2 * p + 1]
        )

    def wait_in(c, p, ra, rb):
        pltpu.make_async_copy(
            x_hbm.at[pl.ds(ra, _L), pl.ds(c * _CC, _CC)], _in_view(2 * p), in_sems[2 * p]
        ).wait()
        pltpu.make_async_copy(
            x_hbm.at[pl.ds(rb, _L), pl.ds(c * _CC, _CC)], _in_view(2 * p + 1), in_sems[2 * p + 1]
        ).wait()

    def start_out(c, p, ra, rb):
        pltpu.async_copy(
            _out_view(2 * p), o_hbm.at[pl.ds(ra, _L), pl.ds(c * _CC, _CC)], out_sems[2 * p]
        )
        pltpu.async_copy(
            _out_view(2 * p + 1), o_hbm.at[pl.ds(rb, _L), pl.ds(c * _CC, _CC)], out_sems[2 * p + 1]
        )

    def wait_out(c, p, ra, rb):
        pltpu.make_async_copy(
            _out_view(2 * p), o_hbm.at[pl.ds(ra, _L), pl.ds(c * _CC, _CC)], out_sems[2 * p]
        ).wait()
        pltpu.make_async_copy(
            _out_view(2 * p + 1), o_hbm.at[pl.ds(rb, _L), pl.ds(c * _CC, _CC)], out_sems[2 * p + 1]
        ).wait()

    def do_pair(pair, _):
        ra = row0 + (2 * pair) * _L
        rb = ra + _L
        start_in(0, 0, ra, rb)
        acc_a = jnp.zeros((_L,), jnp.float32)
        acc_b = jnp.zeros((_L,), jnp.float32)
        for c in range(nch):
            p = c % 2
            if c + 1 < nch:
                start_in(c + 1, 1 - p, ra, rb)
            wait_in(c, p, ra, rb)
            if c >= 2:
                wait_out(c - 2, p, ra, rb)
            acc_a, acc_b = _sc_scan_pair(
                in_bufs[2 * p], in_bufs[2 * p + 1],
                out_bufs[2 * p], out_bufs[2 * p + 1],
                acc_a, acc_b, rows_idx,
            )
            start_out(c, p, ra, rb)
        wait_out(nch - 2, nch % 2, ra, rb)
        wait_out(nch - 1, (nch - 1) % 2, ra, rb)
        return _

    lax.fori_loop(0, rows_per_w // (2 * _L), do_pair, 0)


@jax.jit
def kernel(x):
    rows, cols = x.shape
    mesh = plsc.VectorSubcoreMesh(core_axis_name="c", subcore_axis_name="s")
    buf = pltpu.VMEM((_L, _CCP), jnp.float32)
    return pl.kernel(
        _sc_body,
        out_type=jax.ShapeDtypeStruct((rows, cols), x.dtype),
        mesh=mesh,
        scratch_types=[
            [buf] * 4,
            [buf] * 4,
            [pltpu.SemaphoreType.DMA] * 4,
            [pltpu.SemaphoreType.DMA] * 4,
        ],
        compiler_params=pltpu.CompilerParams(
            use_tc_tiling_on_sc=False, needs_layout_passes=False
        ),
    )(x)


def _tc_cumsum_body(x_ref, o_ref, *, cb: int):
    rb, cols = x_ref.shape
    nchunk = cols // cb
    row = lax.broadcasted_iota(jnp.int32, (cb, cb), 0)
    col = lax.broadcasted_iota(jnp.int32, (cb, cb), 1)
    tri = (row <= col).astype(jnp.float32)

    carry = jnp.zeros((rb, 1), jnp.float32)
    for c in range(nchunk):
        blk = x_ref[:, c * cb : (c + 1) * cb]
        cs = lax.dot(blk, tri, preferred_element_type=jnp.float32)
        o_ref[:, c * cb : (c + 1) * cb] = cs + carry
        carry = carry + cs[:, cb - 1 : cb]


@jax.jit
def kernel_tc(x):
    rows, cols = x.shape
    rb = 256
    cb = 256
    body = functools.partial(_tc_cumsum_body, cb=cb)
    return pl.pallas_call(
        body,
        grid=(rows // rb,),
        in_specs=[pl.BlockSpec((rb, cols), lambda i: (i, 0))],
        out_specs=pl.BlockSpec((rb, cols), lambda i: (i, 0)),
        out_shape=jax.ShapeDtypeStruct((rows, cols), x.dtype),
    )(x)


# speedup vs baseline: 3.0013x; 1.0032x over previous
import functools

import jax
import jax.numpy as jnp
from jax import lax
from jax.experimental import pallas as pl
from jax.experimental.pallas import tpu as pltpu
from jax.experimental.pallas import tpu_sc as plsc

_NC = 2
_NS = 16
_NW = _NC * _NS
_L = 16

_CC = 512
_CCP = _CC + 8
_U = 8


def _sc_scan_pair(in_a, in_b, out_a, out_b, acc_a, acc_b, rows_idx):
    cc = _CC

    def step(i, carry):
        acc_a, acc_b, colv = carry
        for _ in range(_U):
            va = plsc.load_gather(in_a, [rows_idx, colv])
            vb = plsc.load_gather(in_b, [rows_idx, colv])
            acc_a = acc_a + va
            acc_b = acc_b + vb
            plsc.store_scatter(out_a, [rows_idx, colv], acc_a)
            plsc.store_scatter(out_b, [rows_idx, colv], acc_b)
            colv = colv + 1
        return acc_a, acc_b, colv

    colv0 = jnp.zeros((_L,), jnp.int32)
    acc_a, acc_b, _ = lax.fori_loop(0, cc // _U, step, (acc_a, acc_b, colv0))
    return acc_a, acc_b


def _sc_body(x_hbm, o_hbm, in_bufs, out_bufs, in_sems, out_sems):
    rows, cols = x_hbm.shape
    rows_per_w = rows // _NW
    nch = cols // _CC
    wid = lax.axis_index("s") * _NC + lax.axis_index("c")
    row0 = wid * rows_per_w
    rows_idx = lax.iota(jnp.int32, _L)

    def _in_view(i):
        return in_bufs[i].at[:, pl.ds(0, _CC)]

    def _out_view(i):
        return out_bufs[i].at[:, pl.ds(0, _CC)]

    def start_in(c, p, ra, rb):
        pltpu.async_copy(
            x_hbm.at[pl.ds(ra, _L), pl.ds(c * _CC, _CC)], _in_view(2 * p), in_sems[2 * p]
        )
        pltpu.async_copy(
            x_hbm.at[pl.ds(rb, _L), pl.ds(c * _CC, _CC)], _in_view(2 * p + 1), in_sems[2 * p + 1]
        )

    def wait_in(c, p, ra, rb):
        pltpu.make_async_copy(
            x_hbm.at[pl.ds(ra, _L), pl.ds(c * _CC, _CC)], _in_view(2 * p), in_sems[2 * p]
        ).wait()
        pltpu.make_async_copy(
            x_hbm.at[pl.ds(rb, _L), pl.ds(c * _CC, _CC)], _in_view(2 * p + 1), in_sems[2 * p + 1]
        ).wait()

    def start_out(c, p, ra, rb):
        pltpu.async_copy(
            _out_view(2 * p), o_hbm.at[pl.ds(ra, _L), pl.ds(c * _CC, _CC)], out_sems[2 * p]
        )
        pltpu.async_copy(
            _out_view(2 * p + 1), o_hbm.at[pl.ds(rb, _L), pl.ds(c * _CC, _CC)], out_sems[2 * p + 1]
        )

    def wait_out(c, p, ra, rb):
        pltpu.make_async_copy(
            _out_view(2 * p), o_hbm.at[pl.ds(ra, _L), pl.ds(c * _CC, _CC)], out_sems[2 * p]
        ).wait()
        pltpu.make_async_copy(
            _out_view(2 * p + 1), o_hbm.at[pl.ds(rb, _L), pl.ds(c * _CC, _CC)], out_sems[2 * p + 1]
        ).wait()

    def do_pair(pair, _):
        ra = row0 + (2 * pair) * _L
        rb = ra + _L
        start_in(0, 0, ra, rb)
        acc_a = jnp.zeros((_L,), jnp.float32)
        acc_b = jnp.zeros((_L,), jnp.float32)
        for c in range(nch):
            p = c % 2
            if c + 1 < nch:
                start_in(c + 1, 1 - p, ra, rb)
            wait_in(c, p, ra, rb)
            if c >= 2:
                wait_out(c - 2, p, ra, rb)
            acc_a, acc_b = _sc_scan_pair(
                in_bufs[2 * p], in_bufs[2 * p + 1],
                out_bufs[2 * p], out_bufs[2 * p + 1],
                acc_a, acc_b, rows_idx,
            )
            start_out(c, p, ra, rb)
        wait_out(nch - 2, nch % 2, ra, rb)
        wait_out(nch - 1, (nch - 1) % 2, ra, rb)
        return _

    lax.fori_loop(0, rows_per_w // (2 * _L), do_pair, 0)


@jax.jit
def kernel(x):
    rows, cols = x.shape
    mesh = plsc.VectorSubcoreMesh(core_axis_name="c", subcore_axis_name="s")
    buf = pltpu.VMEM((_L, _CCP), jnp.float32)
    return pl.kernel(
        _sc_body,
        out_type=jax.ShapeDtypeStruct((rows, cols), x.dtype),
        mesh=mesh,
        scratch_types=[
            [buf] * 4,
            [buf] * 4,
            [pltpu.SemaphoreType.DMA] * 4,
            [pltpu.SemaphoreType.DMA] * 4,
        ],
        compiler_params=pltpu.CompilerParams(
            use_tc_tiling_on_sc=False, needs_layout_passes=False
        ),
    )(x)


def _tc_cumsum_body(x_ref, o_ref, *, cb: int):
    rb, cols = x_ref.shape
    nchunk = cols // cb
    row = lax.broadcasted_iota(jnp.int32, (cb, cb), 0)
    col = lax.broadcasted_iota(jnp.int32, (cb, cb), 1)
    tri = (row <= col).astype(jnp.float32)

    carry = jnp.zeros((rb, 1), jnp.float32)
    for c in range(nchunk):
        blk = x_ref[:, c * cb : (c + 1) * cb]
        cs = lax.dot(blk, tri, preferred_element_type=jnp.float32)
        o_ref[:, c * cb : (c + 1) * cb] = cs + carry
        carry = carry + cs[:, cb - 1 : cb]


@jax.jit
def kernel_tc(x):
    rows, cols = x.shape
    rb = 256
    cb = 256
    body = functools.partial(_tc_cumsum_body, cb=cb)
    return pl.pallas_call(
        body,
        grid=(rows // rb,),
        in_specs=[pl.BlockSpec((rb, cols), lambda i: (i, 0))],
        out_specs=pl.BlockSpec((rb, cols), lambda i: (i, 0)),
        out_shape=jax.ShapeDtypeStruct((rows, cols), x.dtype),
    )(x)


# speedup vs baseline: 4.2739x; 1.4240x over previous
import functools

import jax
import jax.numpy as jnp
from jax import lax
from jax.experimental import pallas as pl
from jax.experimental.pallas import tpu as pltpu
from jax.experimental.pallas import tpu_sc as plsc

_NC = 2
_NS = 16
_NW = _NC * _NS
_L = 16

_CC = 1024


def _sc_scan_chunk(in_ref, out_ref, carries):
    def step(j, carry):
        new = []
        for r in range(_L):
            v = in_ref[r, pl.ds(j * _L, _L)]
            cs = plsc.cumsum(v) + carry[r]
            out_ref[r, pl.ds(j * _L, _L)] = cs
            new.append(cs[_L - 1])
        return tuple(new)

    return lax.fori_loop(0, _CC // _L, step, carries)


def _sc_body(x_hbm, o_hbm, in_bufs, out_bufs, in_sems, out_sems):
    rows, cols = x_hbm.shape
    rows_per_w = rows // _NW
    nch = cols // _CC
    wid = lax.axis_index("s") * _NC + lax.axis_index("c")
    row0 = wid * rows_per_w

    def start_in(c, p, r0):
        pltpu.async_copy(
            x_hbm.at[pl.ds(r0, _L), pl.ds(c * _CC, _CC)], in_bufs[p], in_sems[p]
        )

    def wait_in(c, p, r0):
        pltpu.make_async_copy(
            x_hbm.at[pl.ds(r0, _L), pl.ds(c * _CC, _CC)], in_bufs[p], in_sems[p]
        ).wait()

    def start_out(c, p, r0):
        pltpu.async_copy(
            out_bufs[p], o_hbm.at[pl.ds(r0, _L), pl.ds(c * _CC, _CC)], out_sems[p]
        )

    def wait_out(c, p, r0):
        pltpu.make_async_copy(
            out_bufs[p], o_hbm.at[pl.ds(r0, _L), pl.ds(c * _CC, _CC)], out_sems[p]
        ).wait()

    def do_group(g, _):
        r0 = row0 + g * _L
        start_in(0, 0, r0)
        carries = (jnp.float32(0.0),) * _L
        for c in range(nch):
            p = c % 2
            if c + 1 < nch:
                start_in(c + 1, 1 - p, r0)
            wait_in(c, p, r0)
            if c >= 2:
                wait_out(c - 2, p, r0)
            carries = _sc_scan_chunk(in_bufs[p], out_bufs[p], carries)
            start_out(c, p, r0)
        wait_out(nch - 2, nch % 2, r0)
        wait_out(nch - 1, (nch - 1) % 2, r0)
        return _

    lax.fori_loop(0, rows_per_w // _L, do_group, 0)


@jax.jit
def kernel(x):
    rows, cols = x.shape
    mesh = plsc.VectorSubcoreMesh(core_axis_name="c", subcore_axis_name="s")
    buf = pltpu.VMEM((_L, _CC), jnp.float32)
    return pl.kernel(
        _sc_body,
        out_type=jax.ShapeDtypeStruct((rows, cols), x.dtype),
        mesh=mesh,
        scratch_types=[
            [buf] * 2,
            [buf] * 2,
            [pltpu.SemaphoreType.DMA] * 2,
            [pltpu.SemaphoreType.DMA] * 2,
        ],
        compiler_params=pltpu.CompilerParams(
            use_tc_tiling_on_sc=False, needs_layout_passes=False
        ),
    )(x)


def _tc_cumsum_body(x_ref, o_ref, *, cb: int):
    rb, cols = x_ref.shape
    nchunk = cols // cb
    row = lax.broadcasted_iota(jnp.int32, (cb, cb), 0)
    col = lax.broadcasted_iota(jnp.int32, (cb, cb), 1)
    tri = (row <= col).astype(jnp.float32)

    carry = jnp.zeros((rb, 1), jnp.float32)
    for c in range(nchunk):
        blk = x_ref[:, c * cb : (c + 1) * cb]
        cs = lax.dot(blk, tri, preferred_element_type=jnp.float32)
        o_ref[:, c * cb : (c + 1) * cb] = cs + carry
        carry = carry + cs[:, cb - 1 : cb]


@jax.jit
def kernel_tc(x):
    rows, cols = x.shape
    rb = 256
    cb = 256
    body = functools.partial(_tc_cumsum_body, cb=cb)
    return pl.pallas_call(
        body,
        grid=(rows // rb,),
        in_specs=[pl.BlockSpec((rb, cols), lambda i: (i, 0))],
        out_specs=pl.BlockSpec((rb, cols), lambda i: (i, 0)),
        out_shape=jax.ShapeDtypeStruct((rows, cols), x.dtype),
    )(x)


# speedup vs baseline: 7.3587x; 1.7218x over previous
import functools

import jax
import jax.numpy as jnp
from jax import lax
from jax.experimental import pallas as pl
from jax.experimental.pallas import tpu as pltpu
from jax.experimental.pallas import tpu_sc as plsc

_NC = 2
_NS = 16
_NW = _NC * _NS
_L = 16

_CC = 1024


def _sc_scan_chunk(in_ref, out_ref, carries):
    def step(j, carry):
        new = []
        for r in range(_L):
            v = in_ref[r, pl.ds(j * _L, _L)]
            cs = plsc.cumsum(v) + carry[r]
            out_ref[r, pl.ds(j * _L, _L)] = cs
            new.append(cs[_L - 1])
        return tuple(new)

    return lax.fori_loop(0, _CC // _L, step, carries)


def _sc_body(x_hbm, o_hbm, in_bufs, out_bufs, in_sems, out_sems, *, x_row0):
    rows, cols = o_hbm.shape
    rows_per_w = rows // _NW
    nch = cols // _CC
    wid = lax.axis_index("s") * _NC + lax.axis_index("c")
    row0 = wid * rows_per_w

    def start_in(c, p, r0):
        pltpu.async_copy(
            x_hbm.at[pl.ds(x_row0 + r0, _L), pl.ds(c * _CC, _CC)], in_bufs[p], in_sems[p]
        )

    def wait_in(c, p, r0):
        pltpu.make_async_copy(
            x_hbm.at[pl.ds(x_row0 + r0, _L), pl.ds(c * _CC, _CC)], in_bufs[p], in_sems[p]
        ).wait()

    def start_out(c, p, r0):
        pltpu.async_copy(
            out_bufs[p], o_hbm.at[pl.ds(r0, _L), pl.ds(c * _CC, _CC)], out_sems[p]
        )

    def wait_out(c, p, r0):
        pltpu.make_async_copy(
            out_bufs[p], o_hbm.at[pl.ds(r0, _L), pl.ds(c * _CC, _CC)], out_sems[p]
        ).wait()

    def do_group(g, _):
        r0 = row0 + g * _L
        start_in(0, 0, r0)
        carries = (jnp.float32(0.0),) * _L
        for c in range(nch):
            p = c % 2
            if c + 1 < nch:
                start_in(c + 1, 1 - p, r0)
            wait_in(c, p, r0)
            if c >= 2:
                wait_out(c - 2, p, r0)
            carries = _sc_scan_chunk(in_bufs[p], out_bufs[p], carries)
            start_out(c, p, r0)
        wait_out(nch - 2, nch % 2, r0)
        wait_out(nch - 1, (nch - 1) % 2, r0)
        return _

    lax.fori_loop(0, rows_per_w // _L, do_group, 0)


def _sc_call(x, out_rows, x_row0):
    cols = x.shape[1]
    mesh = plsc.VectorSubcoreMesh(core_axis_name="c", subcore_axis_name="s")
    buf = pltpu.VMEM((_L, _CC), jnp.float32)
    return pl.kernel(
        functools.partial(_sc_body, x_row0=x_row0),
        out_type=jax.ShapeDtypeStruct((out_rows, cols), x.dtype),
        mesh=mesh,
        scratch_types=[
            [buf] * 2,
            [buf] * 2,
            [pltpu.SemaphoreType.DMA] * 2,
            [pltpu.SemaphoreType.DMA] * 2,
        ],
        compiler_params=pltpu.CompilerParams(
            use_tc_tiling_on_sc=False, needs_layout_passes=False
        ),
    )(x)


_R_SC = 512


@jax.jit
def kernel(x):
    rows, cols = x.shape
    r_tc = rows - _R_SC
    sc_out = _sc_call(x, _R_SC, r_tc)
    tc_out = _tc_call(x, r_tc)
    return lax.dynamic_update_slice(tc_out, sc_out, (r_tc, 0))


def _tc_cumsum_body(x_ref, o_ref, *, cb: int):
    rb, cols = x_ref.shape
    nchunk = cols // cb
    row = lax.broadcasted_iota(jnp.int32, (cb, cb), 0)
    col = lax.broadcasted_iota(jnp.int32, (cb, cb), 1)
    tri = (row <= col).astype(jnp.float32)

    carry = jnp.zeros((rb, 1), jnp.float32)
    for c in range(nchunk):
        blk = x_ref[:, c * cb : (c + 1) * cb]
        cs = lax.dot(blk, tri, preferred_element_type=jnp.float32)
        o_ref[:, c * cb : (c + 1) * cb] = cs + carry
        carry = carry + cs[:, cb - 1 : cb]


def _tc_call(x, r_tc):
    rows, cols = x.shape
    rb = 256
    cb = 256
    body = functools.partial(_tc_cumsum_body, cb=cb)
    return pl.pallas_call(
        body,
        grid=(r_tc // rb,),
        in_specs=[pl.BlockSpec((rb, cols), lambda i: (i, 0))],
        out_specs=pl.BlockSpec((rb, cols), lambda i: (i, 0)),
        out_shape=jax.ShapeDtypeStruct((rows, cols), x.dtype),
    )(x)
